# initial kernel scaffold (unmeasured)
import jax
import jax.numpy as jnp
from jax import lax
from jax.experimental import pallas as pl
from jax.experimental.pallas import tpu as pltpu


def kernel(
    x,
):
    def body(*refs):
        pass

    out_shape = jax.ShapeDtypeStruct(..., jnp.float32)
    return pl.pallas_call(body, out_shape=out_shape)(...)


# baseline (device time: 2223125 ns/iter reference)
import jax
import jax.numpy as jnp
from jax import lax
from jax.experimental import pallas as pl
from jax.experimental.pallas import tpu as pltpu

Z = 4


def kernel(x):
    x = x.astype(jnp.bfloat16)
    m, n = x.shape

    def body(x_ref, out_ref, send_sems, recv_sems, copy_sem):
        my_x = lax.axis_index("x")
        my_y = lax.axis_index("y")
        my_z = lax.axis_index("z")
        right = lax.rem(my_z + 1, Z)
        left = lax.rem(my_z + Z - 1, Z)

        local = pltpu.make_async_copy(
            x_ref, out_ref.at[pl.ds(my_z * m, m), :], copy_sem
        )
        local.start()

        barrier_sem = pltpu.get_barrier_semaphore()
        for nbr in (left, right):
            pl.semaphore_signal(
                barrier_sem,
                inc=1,
                device_id=(my_x, my_y, nbr),
                device_id_type=pl.DeviceIdType.MESH,
            )
        pl.semaphore_wait(barrier_sem, 2)

        local.wait()

        for h in range(Z - 1):
            origin = lax.rem(my_z - h + Z, Z)
            rdma = pltpu.make_async_remote_copy(
                src_ref=out_ref.at[pl.ds(origin * m, m), :],
                dst_ref=out_ref.at[pl.ds(origin * m, m), :],
                send_sem=send_sems.at[h],
                recv_sem=recv_sems.at[h],
                device_id=(my_x, my_y, right),
                device_id_type=pl.DeviceIdType.MESH,
            )
            rdma.start()
            rdma.wait()

    return pl.pallas_call(
        body,
        out_shape=jax.ShapeDtypeStruct((Z * m, n), jnp.bfloat16),
        in_specs=[pl.BlockSpec(memory_space=pl.ANY)],
        out_specs=pl.BlockSpec(memory_space=pl.ANY),
        scratch_shapes=[
            pltpu.SemaphoreType.DMA((Z - 1,)),
            pltpu.SemaphoreType.DMA((Z - 1,)),
            pltpu.SemaphoreType.DMA,
        ],
        compiler_params=pltpu.CompilerParams(collective_id=0),
    )(x)


# device time: 1139512 ns/iter; 1.9509x vs baseline; 1.9509x over previous
import jax
import jax.numpy as jnp
from jax import lax
from jax.experimental import pallas as pl
from jax.experimental.pallas import tpu as pltpu

Z = 4
Q = 4


def kernel(x):
    x = x.astype(jnp.bfloat16)
    m, n = x.shape
    half = m // 2
    sb = half // Q

    def body(x_ref, out_ref, zr_recv, zl_recv, x_recv, zr_send, zl_send,
             x_send, copy_sem):
        my_x = lax.axis_index("x")
        my_y = lax.axis_index("y")
        my_z = lax.axis_index("z")
        half_off = my_x * half
        other_off = (1 - my_x) * half
        has_l = my_z > 0
        has_r = my_z < Z - 1
        partner = (1 - my_x, my_y, my_z)
        left = (my_x, my_y, my_z - 1)
        right = (my_x, my_y, my_z + 1)

        def rc(src_row, dst_row, send_sem, recv_sem, dev):
            return pltpu.make_async_remote_copy(
                src_ref=out_ref.at[pl.ds(src_row, sb), :],
                dst_ref=out_ref.at[pl.ds(dst_row, sb), :],
                send_sem=send_sem,
                recv_sem=recv_sem,
                device_id=dev,
                device_id_type=pl.DeviceIdType.MESH,
            )

        local = pltpu.make_async_copy(
            x_ref, out_ref.at[pl.ds(my_z * m, m), :], copy_sem
        )
        local.start()

        bar = pltpu.get_barrier_semaphore()

        @pl.when(has_l)
        def _():
            pl.semaphore_signal(bar, inc=1, device_id=left,
                                device_id_type=pl.DeviceIdType.MESH)
            pl.semaphore_wait(bar, 1)

        @pl.when(has_r)
        def _():
            pl.semaphore_signal(bar, inc=1, device_id=right,
                                device_id_type=pl.DeviceIdType.MESH)
            pl.semaphore_wait(bar, 1)

        pl.semaphore_signal(bar, inc=1, device_id=partner,
                            device_id_type=pl.DeviceIdType.MESH)
        pl.semaphore_wait(bar, 1)

        for q in range(Q):
            @pl.when(has_r)
            def _(q=q):
                pltpu.make_async_remote_copy(
                    src_ref=x_ref.at[pl.ds(half_off + q * sb, sb), :],
                    dst_ref=out_ref.at[
                        pl.ds(my_z * m + half_off + q * sb, sb), :],
                    send_sem=zr_send,
                    recv_sem=zr_recv.at[q],
                    device_id=right,
                    device_id_type=pl.DeviceIdType.MESH,
                ).start()

            @pl.when(has_l)
            def _(q=q):
                pltpu.make_async_remote_copy(
                    src_ref=x_ref.at[pl.ds(half_off + q * sb, sb), :],
                    dst_ref=out_ref.at[
                        pl.ds(my_z * m + half_off + q * sb, sb), :],
                    send_sem=zl_send,
                    recv_sem=zl_recv.at[q],
                    device_id=left,
                    device_id_type=pl.DeviceIdType.MESH,
                ).start()

        for s in range(Z - 1):
            o_r = my_z - 1 - s
            o_l = my_z + 1 + s
            rv = o_r >= 0
            lv = o_l <= Z - 1
            for q in range(Q):
                row_r = o_r * m + half_off + q * sb
                row_l = o_l * m + half_off + q * sb

                @pl.when(rv)
                def _(s=s, q=q, row_r=row_r):
                    rc(row_r, row_r, zr_send, zr_recv.at[s * Q + q],
                       (my_x, my_y, my_z)).wait_recv()

                if s < Z - 2:
                    @pl.when(rv & has_r)
                    def _(s=s, q=q, row_r=row_r):
                        rc(row_r, row_r, zr_send,
                           zr_recv.at[(s + 1) * Q + q], right).start()

                @pl.when(rv)
                def _(q=q, row_r=row_r):
                    rc(row_r, row_r, x_send, x_recv, partner).start()

                @pl.when(lv)
                def _(s=s, q=q, row_l=row_l):
                    rc(row_l, row_l, zl_send, zl_recv.at[s * Q + q],
                       (my_x, my_y, my_z)).wait_recv()

                if s < Z - 2:
                    @pl.when(lv & has_l)
                    def _(s=s, q=q, row_l=row_l):
                        rc(row_l, row_l, zl_send,
                           zl_recv.at[(s + 1) * Q + q], left).start()

                @pl.when(lv)
                def _(q=q, row_l=row_l):
                    rc(row_l, row_l, x_send, x_recv, partner).start()

        for s in range(Z - 1):
            o_r = my_z - 1 - s
            o_l = my_z + 1 + s
            rv = o_r >= 0
            lv = o_l <= Z - 1
            for q in range(Q):
                @pl.when(rv)
                def _(q=q, row=o_r * m + other_off + q * sb):
                    rc(row, row, x_send, x_recv, partner).wait_recv()

                @pl.when(lv)
                def _(q=q, row=o_l * m + other_off + q * sb):
                    rc(row, row, x_send, x_recv, partner).wait_recv()

        for q in range(Q):
            @pl.when(has_r)
            def _(q=q):
                rc(my_z * m + half_off + q * sb,
                   my_z * m + half_off + q * sb, zr_send,
                   zr_recv.at[q], right).wait_send()

            @pl.when(has_l)
            def _(q=q):
                rc(my_z * m + half_off + q * sb,
                   my_z * m + half_off + q * sb, zl_send,
                   zl_recv.at[q], left).wait_send()

        for s in range(Z - 1):
            o_r = my_z - 1 - s
            o_l = my_z + 1 + s
            rv = o_r >= 0
            lv = o_l <= Z - 1
            for q in range(Q):
                row_r = o_r * m + half_off + q * sb
                row_l = o_l * m + half_off + q * sb
                if s < Z - 2:
                    @pl.when(rv & has_r)
                    def _(s=s, q=q, row_r=row_r):
                        rc(row_r, row_r, zr_send,
                           zr_recv.at[(s + 1) * Q + q], right).wait_send()

                    @pl.when(lv & has_l)
                    def _(s=s, q=q, row_l=row_l):
                        rc(row_l, row_l, zl_send,
                           zl_recv.at[(s + 1) * Q + q], left).wait_send()

                @pl.when(rv)
                def _(q=q, row_r=row_r):
                    rc(row_r, row_r, x_send, x_recv, partner).wait_send()

                @pl.when(lv)
                def _(q=q, row_l=row_l):
                    rc(row_l, row_l, x_send, x_recv, partner).wait_send()

        local.wait()

    return pl.pallas_call(
        body,
        out_shape=jax.ShapeDtypeStruct((Z * m, n), jnp.bfloat16),
        in_specs=[pl.BlockSpec(memory_space=pl.ANY)],
        out_specs=pl.BlockSpec(memory_space=pl.ANY),
        scratch_shapes=[
            pltpu.SemaphoreType.DMA(((Z - 1) * Q,)),
            pltpu.SemaphoreType.DMA(((Z - 1) * Q,)),
            pltpu.SemaphoreType.DMA,
            pltpu.SemaphoreType.DMA,
            pltpu.SemaphoreType.DMA,
            pltpu.SemaphoreType.DMA,
            pltpu.SemaphoreType.DMA,
        ],
        compiler_params=pltpu.CompilerParams(collective_id=0),
    )(x)
